# Initial kernel scaffold; baseline (speedup 1.0000x reference)
import jax
import jax.numpy as jnp
from jax.experimental import pallas as pl


def kernel(x, center, ptr):
    raise NotImplementedError("write your pallas kernel here")


# speedup vs baseline: 24.6516x; 24.6516x over previous
import functools

import jax
import jax.numpy as jnp
from jax import lax
from jax.experimental import pallas as pl
from jax.experimental.pallas import tpu as pltpu
from jax.experimental.pallas import tpu_sc as plsc

TRIGGER = 0.5
N_NODES = 100000
D_FEAT = 128
B = 1024

NUM_CORES = 2
NUM_SUBCORES = 16
LANES = 16
NUM_WORKERS = NUM_CORES * NUM_SUBCORES
CHUNK_ROWS = 160
NUM_CHUNKS = N_NODES // CHUNK_ROWS
CHUNKS_PER_WORKER = -(-NUM_CHUNKS // NUM_WORKERS)
NUM_CP_VREGS = B // LANES

_mesh = plsc.VectorSubcoreMesh(core_axis_name="c", subcore_axis_name="s")


@functools.partial(
    pl.kernel,
    out_type=jax.ShapeDtypeStruct((N_NODES, D_FEAT), jnp.float32),
    mesh=_mesh,
    compiler_params=pltpu.CompilerParams(needs_layout_passes=False),
    scratch_types=[
        pltpu.VMEM((B,), jnp.int32),
        pltpu.VMEM((CHUNK_ROWS, D_FEAT), jnp.float32),
        pltpu.VMEM((2 * LANES,), jnp.int32),
        pltpu.VMEM((2 * LANES,), jnp.int32),
    ],
)
def _sc_copy_update(x_hbm, cp_hbm, out_hbm, cp_v, buf, tmp_c, tmp_m):
    wid = lax.axis_index("s") * NUM_CORES + lax.axis_index("c")

    pltpu.sync_copy(cp_hbm, cp_v)
    tmp_m[pl.ds(0, LANES)] = jnp.zeros((LANES,), jnp.int32)
    tmp_m[pl.ds(LANES, LANES)] = jnp.zeros((LANES,), jnp.int32)

    def chunk_body(k, carry):
        ci = wid + k * NUM_WORKERS

        @pl.when(ci < NUM_CHUNKS)
        def _():
            row0 = ci * CHUNK_ROWS
            pltpu.sync_copy(x_hbm.at[pl.ds(row0, CHUNK_ROWS)], buf)

            lo_v = jnp.full((LANES,), row0, jnp.int32)
            hi_v = jnp.full((LANES,), row0 + CHUNK_ROWS, jnp.int32)
            c_vec0 = cp_v[pl.ds(0, LANES)]
            ones = jnp.ones((LANES,), jnp.int32)
            zeros = jnp.zeros((LANES,), jnp.int32)
            for v in range(NUM_CP_VREGS):
                c_vec = cp_v[pl.ds(v * LANES, LANES)]
                m_ge = jnp.where(c_vec >= lo_v, ones, zeros)
                m_lt = jnp.where(c_vec < hi_v, ones, zeros)
                m_i32 = m_ge * m_lt
                n_match = jnp.sum(m_i32)

                @pl.when(n_match > 0)
                def _():
                    tmp_c[pl.ds(0, LANES)] = c_vec
                    tmp_m[pl.ds(0, LANES)] = m_i32

                    def lane_body(l, carry2):
                        mval = tmp_m[pl.ds(l, LANES)][0]

                        @pl.when(mval > 0)
                        def _():
                            r = tmp_c[pl.ds(l, LANES)][0] - row0
                            s0 = buf[r, pl.ds(0, LANES)]
                            buf[r, pl.ds(0, LANES)] = jnp.minimum(s0 + TRIGGER, 1.0)
                            s1 = buf[r, pl.ds(LANES, LANES)]
                            buf[r, pl.ds(LANES, LANES)] = jnp.minimum(s1 + TRIGGER, 1.0)
                        return carry2

                    lax.fori_loop(0, LANES, lane_body, 0)

            pltpu.sync_copy(buf, out_hbm.at[pl.ds(row0, CHUNK_ROWS)])

        return carry

    lax.fori_loop(0, CHUNKS_PER_WORKER, chunk_body, 0)


def kernel(x, center, ptr):
    cp = (center + ptr[:-1]).astype(jnp.int32)
    return _sc_copy_update(x, cp)


# speedup vs baseline: 75.0880x; 3.0460x over previous
import functools

import jax
import jax.numpy as jnp
from jax import lax
from jax.experimental import pallas as pl
from jax.experimental.pallas import tpu as pltpu
from jax.experimental.pallas import tpu_sc as plsc

TRIGGER = 0.5
N_NODES = 100000
D_FEAT = 128
B = 1024

NUM_CORES = 2
NUM_SUBCORES = 16
LANES = 16
NUM_WORKERS = NUM_CORES * NUM_SUBCORES
CHUNK_ROWS = 400
NUM_CHUNKS = N_NODES // CHUNK_ROWS
CHUNKS_PER_WORKER = -(-NUM_CHUNKS // NUM_WORKERS)
NUM_CP_VREGS = B // LANES

_mesh = plsc.VectorSubcoreMesh(core_axis_name="c", subcore_axis_name="s")


@functools.partial(
    pl.kernel,
    out_type=jax.ShapeDtypeStruct((N_NODES, D_FEAT), jnp.float32),
    mesh=_mesh,
    compiler_params=pltpu.CompilerParams(needs_layout_passes=False),
    scratch_types=[
        pltpu.VMEM((B,), jnp.int32),
        pltpu.VMEM((CHUNK_ROWS, D_FEAT), jnp.float32),
        pltpu.VMEM((CHUNK_ROWS, D_FEAT), jnp.float32),
        pltpu.VMEM((2 * LANES,), jnp.int32),
        pltpu.VMEM((2 * LANES,), jnp.int32),
        pltpu.SemaphoreType.DMA,
        pltpu.SemaphoreType.DMA,
        pltpu.SemaphoreType.DMA,
        pltpu.SemaphoreType.DMA,
    ],
)
def _sc_copy_update(x_hbm, cp_hbm, out_hbm, cp_v, buf0, buf1,
                    tmp_c, tmp_m, isem0, isem1, osem0, osem1):
    wid = lax.axis_index("s") * NUM_CORES + lax.axis_index("c")

    pltpu.sync_copy(cp_hbm, cp_v)
    tmp_m[pl.ds(0, LANES)] = jnp.zeros((LANES,), jnp.int32)
    tmp_m[pl.ds(LANES, LANES)] = jnp.zeros((LANES,), jnp.int32)

    def ci_of(k):
        return wid + k * NUM_WORKERS

    def start_in(k, buf, isem):
        @pl.when(ci_of(k) < NUM_CHUNKS)
        def _():
            row0 = ci_of(k) * CHUNK_ROWS
            pltpu.make_async_copy(
                x_hbm.at[pl.ds(row0, CHUNK_ROWS)], buf, isem).start()

    def wait_in(k, buf, isem):
        @pl.when(ci_of(k) < NUM_CHUNKS)
        def _():
            pltpu.make_async_copy(
                x_hbm.at[pl.ds(0, CHUNK_ROWS)], buf, isem).wait()

    def start_out(k, buf, osem):
        @pl.when(ci_of(k) < NUM_CHUNKS)
        def _():
            row0 = ci_of(k) * CHUNK_ROWS
            pltpu.make_async_copy(
                buf, out_hbm.at[pl.ds(row0, CHUNK_ROWS)], osem).start()

    def wait_out(k, buf, osem):
        @pl.when(ci_of(k) < NUM_CHUNKS)
        def _():
            pltpu.make_async_copy(
                buf, out_hbm.at[pl.ds(0, CHUNK_ROWS)], osem).wait()

    def process(k, buf):
        @pl.when(ci_of(k) < NUM_CHUNKS)
        def _():
            row0 = ci_of(k) * CHUNK_ROWS
            lo_v = jnp.full((LANES,), row0, jnp.int32)
            hi_v = jnp.full((LANES,), row0 + CHUNK_ROWS, jnp.int32)
            ones = jnp.ones((LANES,), jnp.int32)
            zeros = jnp.zeros((LANES,), jnp.int32)

            def scan_body(v, carry):
                c_vec = cp_v[pl.ds(v * LANES, LANES)]
                m_ge = jnp.where(c_vec >= lo_v, ones, zeros)
                m_lt = jnp.where(c_vec < hi_v, ones, zeros)
                m_i32 = m_ge * m_lt
                n_match = jnp.sum(m_i32)

                @pl.when(n_match > 0)
                def _():
                    tmp_c[pl.ds(0, LANES)] = c_vec
                    tmp_m[pl.ds(0, LANES)] = m_i32

                    def lane_body(l, carry2):
                        mval = tmp_m[pl.ds(l, LANES)][0]

                        @pl.when(mval > 0)
                        def _():
                            r = tmp_c[pl.ds(l, LANES)][0] - row0
                            s0 = buf[r, pl.ds(0, LANES)]
                            buf[r, pl.ds(0, LANES)] = jnp.minimum(s0 + TRIGGER, 1.0)
                            s1 = buf[r, pl.ds(LANES, LANES)]
                            buf[r, pl.ds(LANES, LANES)] = jnp.minimum(s1 + TRIGGER, 1.0)
                        return carry2

                    lax.fori_loop(0, LANES, lane_body, 0)

                return carry

            lax.fori_loop(0, NUM_CP_VREGS, scan_body, 0)

    def step(k, buf, isem, osem, nk, nbuf, nisem, nosem, drain_next):
        if drain_next:
            wait_out(nk - 2, nbuf, nosem)
        start_in(nk, nbuf, nisem)
        wait_in(k, buf, isem)
        process(k, buf)
        start_out(k, buf, osem)

    start_in(0, buf0, isem0)

    def body(j, carry):
        k0 = 2 * j
        step(k0, buf0, isem0, osem0, k0 + 1, buf1, isem1, osem1, True)
        step(k0 + 1, buf1, isem1, osem1, k0 + 2, buf0, isem0, osem0, True)
        return carry

    step(0, buf0, isem0, osem0, 1, buf1, isem1, osem1, False)
    step(1, buf1, isem1, osem1, 2, buf0, isem0, osem0, True)
    lax.fori_loop(1, CHUNKS_PER_WORKER // 2, body, 0)
    wait_out(CHUNKS_PER_WORKER - 1, buf1, osem1)


def kernel(x, center, ptr):
    cp = (center + ptr[:-1]).astype(jnp.int32)
    return _sc_copy_update(x, cp)


# speedup vs baseline: 78.8472x; 1.0501x over previous
import functools

import jax
import jax.numpy as jnp
from jax import lax
from jax.experimental import pallas as pl
from jax.experimental.pallas import tpu as pltpu
from jax.experimental.pallas import tpu_sc as plsc

TRIGGER = 0.5
N_NODES = 100000
D_FEAT = 128
B = 1024

NUM_CORES = 2
NUM_SUBCORES = 16
LANES = 16
NUM_WORKERS = NUM_CORES * NUM_SUBCORES
CHUNK_ROWS = 400
NUM_CHUNKS = N_NODES // CHUNK_ROWS
CHUNKS_PER_WORKER = -(-NUM_CHUNKS // NUM_WORKERS)
NUM_CP_VREGS = B // LANES
SLOT_WORDS = NUM_CP_VREGS * LANES + LANES

_mesh = plsc.VectorSubcoreMesh(core_axis_name="c", subcore_axis_name="s")


@functools.partial(
    pl.kernel,
    out_type=jax.ShapeDtypeStruct((N_NODES, D_FEAT), jnp.float32),
    mesh=_mesh,
    compiler_params=pltpu.CompilerParams(needs_layout_passes=False),
    scratch_types=[
        pltpu.VMEM((B,), jnp.int32),
        pltpu.VMEM((CHUNK_ROWS, D_FEAT), jnp.float32),
        pltpu.VMEM((CHUNK_ROWS, D_FEAT), jnp.float32),
        pltpu.VMEM((SLOT_WORDS,), jnp.int32),
        pltpu.VMEM((SLOT_WORDS,), jnp.int32),
        pltpu.SemaphoreType.DMA,
        pltpu.SemaphoreType.DMA,
        pltpu.SemaphoreType.DMA,
        pltpu.SemaphoreType.DMA,
    ],
)
def _sc_copy_update(x_hbm, cp_hbm, out_hbm, cp_v, buf0, buf1,
                    slot_c, slot_m, isem0, isem1, osem0, osem1):
    wid = lax.axis_index("s") * NUM_CORES + lax.axis_index("c")

    pltpu.sync_copy(cp_hbm, cp_v)

    def ci_of(k):
        return wid + k * NUM_WORKERS

    def start_in(k, buf, isem):
        @pl.when(ci_of(k) < NUM_CHUNKS)
        def _():
            row0 = ci_of(k) * CHUNK_ROWS
            pltpu.make_async_copy(
                x_hbm.at[pl.ds(row0, CHUNK_ROWS)], buf, isem).start()

    def wait_in(k, buf, isem):
        @pl.when(ci_of(k) < NUM_CHUNKS)
        def _():
            pltpu.make_async_copy(
                x_hbm.at[pl.ds(0, CHUNK_ROWS)], buf, isem).wait()

    def start_out(k, buf, osem):
        @pl.when(ci_of(k) < NUM_CHUNKS)
        def _():
            row0 = ci_of(k) * CHUNK_ROWS
            pltpu.make_async_copy(
                buf, out_hbm.at[pl.ds(row0, CHUNK_ROWS)], osem).start()

    def wait_out(k, buf, osem):
        @pl.when(ci_of(k) < NUM_CHUNKS)
        def _():
            pltpu.make_async_copy(
                buf, out_hbm.at[pl.ds(0, CHUNK_ROWS)], osem).wait()

    def scan_chunk(k):
        row0 = ci_of(k) * CHUNK_ROWS
        lo_v = jnp.full((LANES,), row0, jnp.int32)
        hi_v = jnp.full((LANES,), row0 + CHUNK_ROWS, jnp.int32)
        ones = jnp.ones((LANES,), jnp.int32)
        zeros = jnp.zeros((LANES,), jnp.int32)

        def scan_body(v, cnt):
            c_vec = cp_v[pl.ds(v * LANES, LANES)]
            m_ge = jnp.where(c_vec >= lo_v, ones, zeros)
            m_lt = jnp.where(c_vec < hi_v, ones, zeros)
            m_i32 = m_ge * m_lt
            n_match = jnp.sum(m_i32)
            slot_c[pl.ds(cnt * LANES, LANES)] = c_vec
            slot_m[pl.ds(cnt * LANES, LANES)] = m_i32
            return cnt + jnp.where(n_match > 0, 1, 0)

        return lax.fori_loop(0, NUM_CP_VREGS, scan_body, 0)

    def apply_chunk(k, cnt, buf):
        @pl.when(ci_of(k) < NUM_CHUNKS)
        def _():
            row0 = ci_of(k) * CHUNK_ROWS

            def slot_body(s, carry):
                def lane_body(l, carry2):
                    off = s * LANES + l
                    mval = slot_m[pl.ds(off, LANES)][0]

                    @pl.when(mval > 0)
                    def _():
                        r = slot_c[pl.ds(off, LANES)][0] - row0
                        s0 = buf[r, pl.ds(0, LANES)]
                        buf[r, pl.ds(0, LANES)] = jnp.minimum(s0 + TRIGGER, 1.0)
                        s1 = buf[r, pl.ds(LANES, LANES)]
                        buf[r, pl.ds(LANES, LANES)] = jnp.minimum(s1 + TRIGGER, 1.0)
                    return carry2

                lax.fori_loop(0, LANES, lane_body, 0)
                return carry

            lax.fori_loop(0, cnt, slot_body, 0)

    def step(k, buf, isem, osem, nk, nbuf, nisem, nosem, drain_next):
        if drain_next:
            wait_out(nk - 2, nbuf, nosem)
        start_in(nk, nbuf, nisem)
        cnt = scan_chunk(k)
        wait_in(k, buf, isem)
        apply_chunk(k, cnt, buf)
        start_out(k, buf, osem)

    start_in(0, buf0, isem0)

    def body(j, carry):
        k0 = 2 * j
        step(k0, buf0, isem0, osem0, k0 + 1, buf1, isem1, osem1, True)
        step(k0 + 1, buf1, isem1, osem1, k0 + 2, buf0, isem0, osem0, True)
        return carry

    step(0, buf0, isem0, osem0, 1, buf1, isem1, osem1, False)
    step(1, buf1, isem1, osem1, 2, buf0, isem0, osem0, True)
    lax.fori_loop(1, CHUNKS_PER_WORKER // 2, body, 0)
    wait_out(CHUNKS_PER_WORKER - 1, buf1, osem1)


def kernel(x, center, ptr):
    cp = (center + ptr[:-1]).astype(jnp.int32)
    return _sc_copy_update(x, cp)


# speedup vs baseline: 80.2022x; 1.0172x over previous
import functools

import jax
import jax.numpy as jnp
from jax import lax
from jax.experimental import pallas as pl
from jax.experimental.pallas import tpu as pltpu
from jax.experimental.pallas import tpu_sc as plsc

TRIGGER = 0.5
N_NODES = 100000
D_FEAT = 128
B = 1024

NUM_CORES = 2
NUM_SUBCORES = 16
LANES = 16
NUM_WORKERS = NUM_CORES * NUM_SUBCORES
CHUNK_ROWS = 200
NUM_CHUNKS = N_NODES // CHUNK_ROWS
CHUNKS_PER_WORKER = -(-NUM_CHUNKS // NUM_WORKERS)
NBUF = 3
NUM_STEPS = -(-CHUNKS_PER_WORKER // NBUF) * NBUF
NUM_CP_VREGS = B // LANES
SLOT_WORDS = NUM_CP_VREGS * LANES + LANES

_mesh = plsc.VectorSubcoreMesh(core_axis_name="c", subcore_axis_name="s")


@functools.partial(
    pl.kernel,
    out_type=jax.ShapeDtypeStruct((N_NODES, D_FEAT), jnp.float32),
    mesh=_mesh,
    compiler_params=pltpu.CompilerParams(needs_layout_passes=False),
    scratch_types=[
        pltpu.VMEM((B,), jnp.int32),
        pltpu.VMEM((CHUNK_ROWS, D_FEAT), jnp.float32),
        pltpu.VMEM((CHUNK_ROWS, D_FEAT), jnp.float32),
        pltpu.VMEM((CHUNK_ROWS, D_FEAT), jnp.float32),
        pltpu.VMEM((SLOT_WORDS,), jnp.int32),
        pltpu.VMEM((SLOT_WORDS,), jnp.int32),
        pltpu.SemaphoreType.DMA,
        pltpu.SemaphoreType.DMA,
        pltpu.SemaphoreType.DMA,
        pltpu.SemaphoreType.DMA,
        pltpu.SemaphoreType.DMA,
        pltpu.SemaphoreType.DMA,
    ],
)
def _sc_copy_update(x_hbm, cp_hbm, out_hbm, cp_v, buf0, buf1, buf2,
                    slot_c, slot_m, isem0, isem1, isem2, osem0, osem1, osem2):
    wid = lax.axis_index("s") * NUM_CORES + lax.axis_index("c")

    pltpu.sync_copy(cp_hbm, cp_v)

    slots = ((buf0, isem0, osem0), (buf1, isem1, osem1), (buf2, isem2, osem2))

    def ci_of(k):
        return wid + k * NUM_WORKERS

    def valid(k):
        ci = ci_of(k)
        return (k >= 0) & (ci < NUM_CHUNKS)

    def start_in(k, buf, isem):
        @pl.when(valid(k))
        def _():
            row0 = ci_of(k) * CHUNK_ROWS
            pltpu.make_async_copy(
                x_hbm.at[pl.ds(row0, CHUNK_ROWS)], buf, isem).start()

    def wait_in(k, buf, isem):
        @pl.when(valid(k))
        def _():
            pltpu.make_async_copy(
                x_hbm.at[pl.ds(0, CHUNK_ROWS)], buf, isem).wait()

    def start_out(k, buf, osem):
        @pl.when(valid(k))
        def _():
            row0 = ci_of(k) * CHUNK_ROWS
            pltpu.make_async_copy(
                buf, out_hbm.at[pl.ds(row0, CHUNK_ROWS)], osem).start()

    def wait_out(k, buf, osem):
        @pl.when(valid(k))
        def _():
            pltpu.make_async_copy(
                buf, out_hbm.at[pl.ds(0, CHUNK_ROWS)], osem).wait()

    def scan_chunk(k):
        row0 = ci_of(k) * CHUNK_ROWS
        lo_v = jnp.full((LANES,), row0, jnp.int32)
        hi_v = jnp.full((LANES,), row0 + CHUNK_ROWS, jnp.int32)
        ones = jnp.ones((LANES,), jnp.int32)
        zeros = jnp.zeros((LANES,), jnp.int32)

        def scan_body(v, cnt):
            c_vec = cp_v[pl.ds(v * LANES, LANES)]
            m_ge = jnp.where(c_vec >= lo_v, ones, zeros)
            m_lt = jnp.where(c_vec < hi_v, ones, zeros)
            m_i32 = m_ge * m_lt
            n_match = jnp.sum(m_i32)
            slot_c[pl.ds(cnt * LANES, LANES)] = c_vec
            slot_m[pl.ds(cnt * LANES, LANES)] = m_i32
            return cnt + jnp.where(n_match > 0, 1, 0)

        return lax.fori_loop(0, NUM_CP_VREGS, scan_body, 0)

    def apply_chunk(k, cnt, buf):
        @pl.when(valid(k))
        def _():
            row0 = ci_of(k) * CHUNK_ROWS

            def slot_body(s, carry):
                def lane_body(l, carry2):
                    off = s * LANES + l
                    mval = slot_m[pl.ds(off, LANES)][0]

                    @pl.when(mval > 0)
                    def _():
                        r = slot_c[pl.ds(off, LANES)][0] - row0
                        s0 = buf[r, pl.ds(0, LANES)]
                        buf[r, pl.ds(0, LANES)] = jnp.minimum(s0 + TRIGGER, 1.0)
                        s1 = buf[r, pl.ds(LANES, LANES)]
                        buf[r, pl.ds(LANES, LANES)] = jnp.minimum(s1 + TRIGGER, 1.0)
                    return carry2

                lax.fori_loop(0, LANES, lane_body, 0)
                return carry

            lax.fori_loop(0, cnt, slot_body, 0)

    def step(k, t):
        buf, isem, osem = slots[t]
        nbuf, nisem, nosem = slots[(t + 1) % NBUF]
        wait_out(k + 1 - NBUF, nbuf, nosem)
        start_in(k + 1, nbuf, nisem)
        cnt = scan_chunk(k)
        wait_in(k, buf, isem)
        apply_chunk(k, cnt, buf)
        start_out(k, buf, osem)

    start_in(0, buf0, isem0)

    def body(j, carry):
        k0 = NBUF * j
        for t in range(NBUF):
            step(k0 + t, t)
        return carry

    lax.fori_loop(0, NUM_STEPS // NBUF, body, 0)


def kernel(x, center, ptr):
    cp = (center + ptr[:-1]).astype(jnp.int32)
    return _sc_copy_update(x, cp)


# speedup vs baseline: 80.2722x; 1.0009x over previous
import functools

import jax
import jax.numpy as jnp
from jax import lax
from jax.experimental import pallas as pl
from jax.experimental.pallas import tpu as pltpu
from jax.experimental.pallas import tpu_sc as plsc

TRIGGER = 0.5
N_NODES = 100000
D_FEAT = 128
B = 1024

NUM_CORES = 2
NUM_SUBCORES = 16
LANES = 16
NUM_WORKERS = NUM_CORES * NUM_SUBCORES
CHUNK_ROWS = 200
CHUNK_ELEMS = CHUNK_ROWS * D_FEAT
NUM_CHUNKS = N_NODES // CHUNK_ROWS
CHUNKS_PER_WORKER = -(-NUM_CHUNKS // NUM_WORKERS)
NBUF = 3
NUM_STEPS = -(-CHUNKS_PER_WORKER // NBUF) * NBUF
NUM_CP_VREGS = B // LANES
SLOT_WORDS = NUM_CP_VREGS * LANES + LANES

_mesh = plsc.VectorSubcoreMesh(core_axis_name="c", subcore_axis_name="s")


@functools.partial(
    pl.kernel,
    out_type=jax.ShapeDtypeStruct((N_NODES * D_FEAT,), jnp.float32),
    mesh=_mesh,
    compiler_params=pltpu.CompilerParams(needs_layout_passes=False),
    scratch_types=[
        pltpu.VMEM((B,), jnp.int32),
        pltpu.VMEM((CHUNK_ELEMS,), jnp.float32),
        pltpu.VMEM((CHUNK_ELEMS,), jnp.float32),
        pltpu.VMEM((CHUNK_ELEMS,), jnp.float32),
        pltpu.VMEM((SLOT_WORDS,), jnp.int32),
        pltpu.VMEM((SLOT_WORDS,), jnp.int32),
        pltpu.SemaphoreType.DMA,
        pltpu.SemaphoreType.DMA,
        pltpu.SemaphoreType.DMA,
        pltpu.SemaphoreType.DMA,
        pltpu.SemaphoreType.DMA,
        pltpu.SemaphoreType.DMA,
    ],
)
def _sc_copy_update(x_hbm, cp_hbm, out_hbm, cp_v, buf0, buf1, buf2,
                    slot_c, slot_m, isem0, isem1, isem2, osem0, osem1, osem2):
    wid = lax.axis_index("s") * NUM_CORES + lax.axis_index("c")

    pltpu.sync_copy(cp_hbm, cp_v)

    slots = ((buf0, isem0, osem0), (buf1, isem1, osem1), (buf2, isem2, osem2))

    def ci_of(k):
        return wid + k * NUM_WORKERS

    def valid(k):
        ci = ci_of(k)
        return (k >= 0) & (ci < NUM_CHUNKS)

    def start_in(k, buf, isem):
        @pl.when(valid(k))
        def _():
            e0 = ci_of(k) * CHUNK_ELEMS
            pltpu.make_async_copy(
                x_hbm.at[pl.ds(e0, CHUNK_ELEMS)], buf, isem).start()

    def wait_in(k, buf, isem):
        @pl.when(valid(k))
        def _():
            pltpu.make_async_copy(
                x_hbm.at[pl.ds(0, CHUNK_ELEMS)], buf, isem).wait()

    def start_out(k, buf, osem):
        @pl.when(valid(k))
        def _():
            e0 = ci_of(k) * CHUNK_ELEMS
            pltpu.make_async_copy(
                buf, out_hbm.at[pl.ds(e0, CHUNK_ELEMS)], osem).start()

    def wait_out(k, buf, osem):
        @pl.when(valid(k))
        def _():
            pltpu.make_async_copy(
                buf, out_hbm.at[pl.ds(0, CHUNK_ELEMS)], osem).wait()

    def scan_chunk(k):
        row0 = ci_of(k) * CHUNK_ROWS
        lo_v = jnp.full((LANES,), row0, jnp.int32)
        hi_v = jnp.full((LANES,), row0 + CHUNK_ROWS, jnp.int32)
        ones = jnp.ones((LANES,), jnp.int32)
        zeros = jnp.zeros((LANES,), jnp.int32)

        def scan_body(v, cnt):
            c_vec = cp_v[pl.ds(v * LANES, LANES)]
            m_ge = jnp.where(c_vec >= lo_v, ones, zeros)
            m_lt = jnp.where(c_vec < hi_v, ones, zeros)
            m_i32 = m_ge * m_lt
            n_match = jnp.sum(m_i32)
            slot_c[pl.ds(cnt * LANES, LANES)] = c_vec
            slot_m[pl.ds(cnt * LANES, LANES)] = m_i32
            return cnt + jnp.where(n_match > 0, 1, 0)

        return lax.fori_loop(0, NUM_CP_VREGS, scan_body, 0)

    def apply_chunk(k, cnt, buf):
        @pl.when(valid(k))
        def _():
            row0 = ci_of(k) * CHUNK_ROWS

            def slot_body(s, carry):
                def lane_body(l, carry2):
                    off = s * LANES + l
                    mval = slot_m[pl.ds(off, LANES)][0]

                    @pl.when(mval > 0)
                    def _():
                        r = slot_c[pl.ds(off, LANES)][0] - row0
                        e = r * D_FEAT
                        s0 = buf[pl.ds(e, LANES)]
                        buf[pl.ds(e, LANES)] = jnp.minimum(s0 + TRIGGER, 1.0)
                        s1 = buf[pl.ds(e + LANES, LANES)]
                        buf[pl.ds(e + LANES, LANES)] = jnp.minimum(s1 + TRIGGER, 1.0)
                    return carry2

                lax.fori_loop(0, LANES, lane_body, 0)
                return carry

            lax.fori_loop(0, cnt, slot_body, 0)

    def step(k, t):
        buf, isem, osem = slots[t]
        nbuf, nisem, nosem = slots[(t + 1) % NBUF]
        wait_out(k + 1 - NBUF, nbuf, nosem)
        start_in(k + 1, nbuf, nisem)
        cnt = scan_chunk(k)
        wait_in(k, buf, isem)
        apply_chunk(k, cnt, buf)
        start_out(k, buf, osem)

    start_in(0, buf0, isem0)

    def body(j, carry):
        k0 = NBUF * j
        for t in range(NBUF):
            step(k0 + t, t)
        return carry

    lax.fori_loop(0, NUM_STEPS // NBUF, body, 0)


def kernel(x, center, ptr):
    cp = (center + ptr[:-1]).astype(jnp.int32)
    flat = _sc_copy_update(x.reshape(N_NODES * D_FEAT), cp)
    return flat.reshape(N_NODES, D_FEAT)
